# baseline (speedup 1.0000x reference)
import functools
import math

import jax
import jax.numpy as jnp
from jax import lax
from jax.experimental import pallas as pl
from jax.experimental.pallas import tpu as pltpu
from jax.experimental.pallas import tpu_sc as plsc

VOCAB = 1000000
EMB = 64
SCALE = math.sqrt(EMB)
BLK = 128
NCHUNK = VOCAB // 128
PAIR_ROWS = VOCAB // 2


def _make_table_pack():
    info = plsc.get_sparse_core_info()
    nc, ns = info.num_cores, info.num_subcores
    nw = nc * ns
    slots = (NCHUNK + nw - 1) // nw + 1
    slots += (-slots) % 4
    mesh = plsc.VectorSubcoreMesh(core_axis_name="c", subcore_axis_name="s")

    @functools.partial(
        pl.kernel,
        out_type=jax.ShapeDtypeStruct((PAIR_ROWS, 128), jnp.float32),
        mesh=mesh,
        scratch_types=[
            pltpu.VMEM((4, EMB, 128), jnp.float32),
            pltpu.VMEM((2, EMB, 128), jnp.float32),
            pltpu.SemaphoreType.DMA,
            pltpu.SemaphoreType.DMA,
            pltpu.SemaphoreType.DMA,
            pltpu.SemaphoreType.DMA,
            pltpu.SemaphoreType.DMA,
            pltpu.SemaphoreType.DMA,
        ],
        compiler_params=pltpu.CompilerParams(needs_layout_passes=False),
    )
    def pack(tab_hbm, tail_hbm, out_hbm, ibuf, obuf,
             isem0, isem1, isem2, isem3, osem0, osem1):
        isem = (isem0, isem1, isem2, isem3)
        osem = (osem0, osem1)
        wid = lax.axis_index("s") * nc + lax.axis_index("c")
        iota16 = lax.iota(jnp.int32, 16)


        def start_in(m, b):
            c = wid + m * nw

            @pl.when(c < NCHUNK)
            def _():
                pltpu.async_copy(
                    tab_hbm.at[:, pl.ds(c * 128, 128)], ibuf.at[b], isem[b]
                )

        def wait_in(m, b):
            c = wid + m * nw
            pltpu.make_async_copy(
                tab_hbm.at[:, pl.ds(c * 128, 128)], ibuf.at[b], isem[b]
            ).wait()

        for q in range(4):
            start_in(q, q)

        vvs = [[lax.add(lax.broadcast(2 * q0 + h, (16,)),
                        lax.mul(iota16, 2)) for h in range(2)]
               for q0 in range(0, EMB, 16)]
        qvs = [lax.add(lax.broadcast(q0, (16,)), iota16)
               for q0 in range(0, EMB, 16)]

        def quad_body(p, _):
            for b in range(4):
                m = 4 * p + b
                c = wid + m * nw
                t = b % 2

                @pl.when(c < NCHUNK)
                def _():
                    wait_in(m, b)

                    if b >= 2:
                        pltpu.make_async_copy(
                            obuf.at[t],
                            out_hbm.at[pl.ds(c * EMB, EMB)], osem[t]
                        ).wait()
                    else:
                        @pl.when(p > 0)
                        def _():
                            pltpu.make_async_copy(
                                obuf.at[t],
                                out_hbm.at[pl.ds(c * EMB, EMB)], osem[t]
                            ).wait()

                    @plsc.parallel_loop(0, EMB, 1, unroll=16)
                    def d_body(d):
                        ev = lax.bitwise_and(
                            lax.add(lax.broadcast(d, (16,)), iota16), EMB - 1)
                        for qi in range(EMB // 16):
                            for h in range(2):
                                vals = plsc.load_gather(
                                    ibuf.at[b], [ev, vvs[qi][h]])
                                lv = lax.add(ev, lax.broadcast(h * EMB, (16,)))
                                plsc.store_scatter(
                                    obuf.at[t], [qvs[qi], lv], vals * SCALE)

                    start_in(m + 4, b)
                    pltpu.async_copy(
                        obuf.at[t], out_hbm.at[pl.ds(c * EMB, EMB)], osem[t]
                    )
            return 0

        lax.fori_loop(0, slots // 4, quad_body, 0)

        @pl.when(wid == 0)
        def _():
            pltpu.sync_copy(tail_hbm, out_hbm.at[pl.ds(PAIR_ROWS - 32, 32)])

        for b in range(2):
            m = slots - 2 + b

            def last_c(mm):
                return wid + mm * nw

            @pl.when(last_c(m) < NCHUNK)
            def _():
                pltpu.make_async_copy(
                    obuf.at[b], out_hbm.at[pl.ds(last_c(m) * EMB, EMB)],
                    osem[b]
                ).wait()

            @pl.when(last_c(m) >= NCHUNK)
            def _():
                @pl.when(last_c(m - 2) < NCHUNK)
                def _():
                    pltpu.make_async_copy(
                        obuf.at[b],
                        out_hbm.at[pl.ds(last_c(m - 2) * EMB, EMB)], osem[b]
                    ).wait()

                @pl.when(last_c(m - 2) >= NCHUNK)
                def _():
                    @pl.when(last_c(m - 4) < NCHUNK)
                    def _():
                        pltpu.make_async_copy(
                            obuf.at[b],
                            out_hbm.at[pl.ds(last_c(m - 4) * EMB, EMB)],
                            osem[b]
                        ).wait()

    return pack


def _make_sc_embed(batch: int, seq: int):
    info = plsc.get_sparse_core_info()
    nc, ns = info.num_cores, info.num_subcores
    nw = nc * ns
    assert batch == BLK * nw and seq % 4 == 0
    mesh = plsc.VectorSubcoreMesh(core_axis_name="c", subcore_axis_name="s")

    @functools.partial(
        pl.kernel,
        out_type=jax.ShapeDtypeStruct((seq, EMB, batch), jnp.float32),
        mesh=mesh,
        scratch_types=[
            pltpu.VMEM((seq, BLK), jnp.int32),
            pltpu.VMEM((5, BLK), jnp.int32),
            pltpu.VMEM((5, BLK, 128), jnp.float32),
            pltpu.VMEM((2, EMB, BLK), jnp.float32),
            pltpu.SemaphoreType.DMA,
            pltpu.SemaphoreType.DMA,
            pltpu.SemaphoreType.DMA,
            pltpu.SemaphoreType.DMA,
            pltpu.SemaphoreType.DMA,
            pltpu.SemaphoreType.DMA,
            pltpu.SemaphoreType.DMA,
        ],
        compiler_params=pltpu.CompilerParams(needs_layout_passes=False),
    )
    def sc_embed(tok_hbm, tab_hbm, out_hbm, idx_v, idxg_v, rows_v, tr_v,
                 gsem0, gsem1, gsem2, gsem3, gsem4, osem0, osem1):
        gsem = (gsem0, gsem1, gsem2, gsem3, gsem4)
        osem = (osem0, osem1)
        wid = lax.axis_index("s") * nc + lax.axis_index("c")
        b0 = wid * BLK
        pltpu.sync_copy(tok_hbm.at[:, pl.ds(b0, BLK)], idx_v)

        iota16 = lax.iota(jnp.int32, 16)

        def fill_idxg(s, slot):
            for j in range(BLK // 16):
                i0 = j * 16
                tv = idx_v[s, pl.ds(i0, 16)]
                idxg_v[slot, pl.ds(i0, 16)] = lax.shift_right_logical(tv, 1)

        def start_gather(s, b):
            pltpu.async_copy(tab_hbm.at[idxg_v.at[b]], rows_v.at[b], gsem[b])

        for q in range(5):
            fill_idxg(q, q)
            start_gather(q, q)

        def dec_body(p, _):
            for k in range(10):
                s = 10 * p + k
                b = k % 5
                t = k % 2

                if k >= 2:
                    pltpu.make_async_copy(
                        tr_v.at[t], out_hbm.at[s, :, pl.ds(b0, BLK)], osem[t]
                    ).wait()
                else:
                    @pl.when(p > 0)
                    def _():
                        pltpu.make_async_copy(
                            tr_v.at[t], out_hbm.at[s, :, pl.ds(b0, BLK)],
                            osem[t]
                        ).wait()

                pltpu.make_async_copy(
                    tab_hbm.at[idxg_v.at[b]], rows_v.at[b], gsem[b]
                ).wait()

                tvs = [idx_v[s, pl.ds(j * 16, 16)] for j in range(BLK // 16)]
                ev0s = [lax.mul(lax.bitwise_and(tv, 1), 64) for tv in tvs]
                ivs = [lax.add(lax.broadcast(j * 16, (16,)), iota16)
                       for j in range(BLK // 16)]

                @plsc.parallel_loop(0, EMB, 1, unroll=16)
                def e_body(e):
                    eb = lax.broadcast(e, (16,))
                    ev = lax.bitwise_and(lax.add(eb, iota16), EMB - 1)
                    for j in range(BLK // 16):
                        col = lax.add(ev, ev0s[j])
                        vals = plsc.load_gather(rows_v.at[b], [ivs[j], col])
                        plsc.store_scatter(tr_v.at[t], [ev, ivs[j]], vals)

                @pl.when(s < seq - 5)
                def _():
                    fill_idxg(s + 5, b)
                    pltpu.async_copy(
                        tab_hbm.at[idxg_v.at[b]], rows_v.at[b], gsem[b]
                    )

                pltpu.async_copy(
                    tr_v.at[t], out_hbm.at[s, :, pl.ds(b0, BLK)], osem[t]
                )
            return 0

        lax.fori_loop(0, seq // 10, dec_body, 0)

        for t in range(2):
            s = seq - 2 + t
            pltpu.make_async_copy(
                tr_v.at[t], out_hbm.at[s, :, pl.ds(b0, BLK)], osem[t]
            ).wait()

    return sc_embed


@jax.jit
def kernel(tokens, table):
    batch, seq = tokens.shape
    tok_t = jnp.transpose(tokens.astype(jnp.int32))
    tab_t = jnp.transpose(table)
    tail2 = (table[VOCAB - 64:] * SCALE).reshape(32, 128)
    packed = _make_table_pack()(tab_t, tail2)
    out3d = _make_sc_embed(batch, seq)(tok_t, packed)
    return jnp.transpose(out3d, (2, 0, 1))


# speedup vs baseline: 1.0152x; 1.0152x over previous
import functools
import math

import jax
import jax.numpy as jnp
from jax import lax
from jax.experimental import pallas as pl
from jax.experimental.pallas import tpu as pltpu
from jax.experimental.pallas import tpu_sc as plsc

VOCAB = 1000000
EMB = 64
SCALE = math.sqrt(EMB)
BLK = 128
NCHUNK = VOCAB // 128
PAIR_ROWS = VOCAB // 2


def _make_table_pack():
    info = plsc.get_sparse_core_info()
    nc, ns = info.num_cores, info.num_subcores
    nw = nc * ns
    slots = (NCHUNK + nw - 1) // nw + 1
    slots += (-slots) % 4
    mesh = plsc.VectorSubcoreMesh(core_axis_name="c", subcore_axis_name="s")

    @functools.partial(
        pl.kernel,
        out_type=jax.ShapeDtypeStruct((PAIR_ROWS, 128), jnp.float32),
        mesh=mesh,
        scratch_types=[
            pltpu.VMEM((4, EMB, 128), jnp.float32),
            pltpu.VMEM((2, EMB, 128), jnp.float32),
            pltpu.SemaphoreType.DMA,
            pltpu.SemaphoreType.DMA,
            pltpu.SemaphoreType.DMA,
            pltpu.SemaphoreType.DMA,
            pltpu.SemaphoreType.DMA,
            pltpu.SemaphoreType.DMA,
        ],
        compiler_params=pltpu.CompilerParams(needs_layout_passes=False),
    )
    def pack(tab_hbm, tail_hbm, out_hbm, ibuf, obuf,
             isem0, isem1, isem2, isem3, osem0, osem1):
        isem = (isem0, isem1, isem2, isem3)
        osem = (osem0, osem1)
        wid = lax.axis_index("s") * nc + lax.axis_index("c")
        iota16 = lax.iota(jnp.int32, 16)


        def start_in(m, b):
            c = wid + m * nw

            @pl.when(c < NCHUNK)
            def _():
                pltpu.async_copy(
                    tab_hbm.at[:, pl.ds(c * 128, 128)], ibuf.at[b], isem[b]
                )

        def wait_in(m, b):
            c = wid + m * nw
            pltpu.make_async_copy(
                tab_hbm.at[:, pl.ds(c * 128, 128)], ibuf.at[b], isem[b]
            ).wait()

        for q in range(4):
            start_in(q, q)

        vvs = [[lax.add(lax.broadcast(2 * q0 + h, (16,)),
                        lax.mul(iota16, 2)) for h in range(2)]
               for q0 in range(0, EMB, 16)]
        qvs = [lax.add(lax.broadcast(q0, (16,)), iota16)
               for q0 in range(0, EMB, 16)]

        def quad_body(p, _):
            for b in range(4):
                m = 4 * p + b
                c = wid + m * nw
                t = b % 2

                @pl.when(c < NCHUNK)
                def _():
                    wait_in(m, b)

                    if b >= 2:
                        pltpu.make_async_copy(
                            obuf.at[t],
                            out_hbm.at[pl.ds(c * EMB, EMB)], osem[t]
                        ).wait()
                    else:
                        @pl.when(p > 0)
                        def _():
                            pltpu.make_async_copy(
                                obuf.at[t],
                                out_hbm.at[pl.ds(c * EMB, EMB)], osem[t]
                            ).wait()

                    @plsc.parallel_loop(0, EMB, 1, unroll=16)
                    def d_body(d):
                        ev = lax.bitwise_and(
                            lax.add(lax.broadcast(d, (16,)), iota16), EMB - 1)
                        for qi in range(EMB // 16):
                            for h in range(2):
                                vals = plsc.load_gather(
                                    ibuf.at[b], [ev, vvs[qi][h]])
                                lv = lax.add(ev, lax.broadcast(h * EMB, (16,)))
                                plsc.store_scatter(
                                    obuf.at[t], [qvs[qi], lv], vals * SCALE)

                    start_in(m + 4, b)
                    pltpu.async_copy(
                        obuf.at[t], out_hbm.at[pl.ds(c * EMB, EMB)], osem[t]
                    )
            return 0

        lax.fori_loop(0, slots // 4, quad_body, 0)

        @pl.when(wid == 0)
        def _():
            pltpu.sync_copy(tail_hbm, out_hbm.at[pl.ds(PAIR_ROWS - 32, 32)])

        for b in range(2):
            m = slots - 2 + b

            def last_c(mm):
                return wid + mm * nw

            @pl.when(last_c(m) < NCHUNK)
            def _():
                pltpu.make_async_copy(
                    obuf.at[b], out_hbm.at[pl.ds(last_c(m) * EMB, EMB)],
                    osem[b]
                ).wait()

            @pl.when(last_c(m) >= NCHUNK)
            def _():
                @pl.when(last_c(m - 2) < NCHUNK)
                def _():
                    pltpu.make_async_copy(
                        obuf.at[b],
                        out_hbm.at[pl.ds(last_c(m - 2) * EMB, EMB)], osem[b]
                    ).wait()

                @pl.when(last_c(m - 2) >= NCHUNK)
                def _():
                    @pl.when(last_c(m - 4) < NCHUNK)
                    def _():
                        pltpu.make_async_copy(
                            obuf.at[b],
                            out_hbm.at[pl.ds(last_c(m - 4) * EMB, EMB)],
                            osem[b]
                        ).wait()

    return pack


def _make_sc_embed(batch: int, seq: int):
    info = plsc.get_sparse_core_info()
    nc, ns = info.num_cores, info.num_subcores
    nw = nc * ns
    assert batch == BLK * nw and seq % 4 == 0
    mesh = plsc.VectorSubcoreMesh(core_axis_name="c", subcore_axis_name="s")

    @functools.partial(
        pl.kernel,
        out_type=jax.ShapeDtypeStruct((seq, EMB, batch), jnp.float32),
        mesh=mesh,
        scratch_types=[
            pltpu.VMEM((seq, BLK), jnp.int32),
            pltpu.VMEM((4, BLK), jnp.int32),
            pltpu.VMEM((4, BLK, 128), jnp.float32),
            pltpu.VMEM((2, EMB, BLK), jnp.float32),
            pltpu.SemaphoreType.DMA,
            pltpu.SemaphoreType.DMA,
            pltpu.SemaphoreType.DMA,
            pltpu.SemaphoreType.DMA,
            pltpu.SemaphoreType.DMA,
            pltpu.SemaphoreType.DMA,
        ],
        compiler_params=pltpu.CompilerParams(needs_layout_passes=False),
    )
    def sc_embed(tok_hbm, tab_hbm, out_hbm, idx_v, idxg_v, rows_v, tr_v,
                 gsem0, gsem1, gsem2, gsem3, osem0, osem1):
        gsem = (gsem0, gsem1, gsem2, gsem3)
        osem = (osem0, osem1)
        wid = lax.axis_index("s") * nc + lax.axis_index("c")
        b0 = wid * BLK
        pltpu.sync_copy(tok_hbm.at[:, pl.ds(b0, BLK)], idx_v)

        iota16 = lax.iota(jnp.int32, 16)

        def fill_idxg(s, slot):
            for j in range(BLK // 16):
                i0 = j * 16
                tv = idx_v[s, pl.ds(i0, 16)]
                idxg_v[slot, pl.ds(i0, 16)] = lax.shift_right_logical(tv, 1)

        def start_gather(s, b):
            pltpu.async_copy(tab_hbm.at[idxg_v.at[b]], rows_v.at[b], gsem[b])

        for q in range(4):
            fill_idxg(q, q)
            start_gather(q, q)

        def quad_body(p, _):
            for b in range(4):
                s = 4 * p + b
                t = b % 2

                if b >= 2:
                    pltpu.make_async_copy(
                        tr_v.at[t], out_hbm.at[s, :, pl.ds(b0, BLK)], osem[t]
                    ).wait()
                else:
                    @pl.when(p > 0)
                    def _():
                        pltpu.make_async_copy(
                            tr_v.at[t], out_hbm.at[s, :, pl.ds(b0, BLK)],
                            osem[t]
                        ).wait()

                pltpu.make_async_copy(
                    tab_hbm.at[idxg_v.at[b]], rows_v.at[b], gsem[b]
                ).wait()

                tvs = [idx_v[s, pl.ds(j * 16, 16)] for j in range(BLK // 16)]
                ev0s = [lax.mul(lax.bitwise_and(tv, 1), 64) for tv in tvs]
                ivs = [lax.add(lax.broadcast(j * 16, (16,)), iota16)
                       for j in range(BLK // 16)]

                @plsc.parallel_loop(0, EMB, 1, unroll=16)
                def e_body(e):
                    eb = lax.broadcast(e, (16,))
                    ev = lax.bitwise_and(lax.add(eb, iota16), EMB - 1)
                    for j in range(BLK // 16):
                        col = lax.add(ev, ev0s[j])
                        vals = plsc.load_gather(rows_v.at[b], [ivs[j], col])
                        plsc.store_scatter(tr_v.at[t], [ev, ivs[j]], vals)

                @pl.when(s < seq - 4)
                def _():
                    fill_idxg(s + 4, b)
                    pltpu.async_copy(
                        tab_hbm.at[idxg_v.at[b]], rows_v.at[b], gsem[b]
                    )

                pltpu.async_copy(
                    tr_v.at[t], out_hbm.at[s, :, pl.ds(b0, BLK)], osem[t]
                )
            return 0

        lax.fori_loop(0, seq // 4, quad_body, 0)

        for t in range(2):
            s = seq - 2 + t
            pltpu.make_async_copy(
                tr_v.at[t], out_hbm.at[s, :, pl.ds(b0, BLK)], osem[t]
            ).wait()

    return sc_embed


@jax.jit
def kernel(tokens, table):
    batch, seq = tokens.shape
    tok_t = jnp.transpose(tokens.astype(jnp.int32))
    tab_t = jnp.transpose(table)
    tail2 = (table[VOCAB - 64:] * SCALE).reshape(32, 128)
    packed = _make_table_pack()(tab_t, tail2)
    out3d = _make_sc_embed(batch, seq)(tok_t, packed)
    return jnp.transpose(out3d, (2, 0, 1))
